# baseline (speedup 1.0000x reference)
import functools

import jax
import jax.numpy as jnp
from jax import lax
from jax.experimental import pallas as pl
from jax.experimental.pallas import tpu as pltpu
from jax.experimental.pallas import tpu_sc as plsc

DIM = 16
MAX_POWER = 99
N = 3
OVERFLOW = (MAX_POWER + 1) ** N
BLK = 128
SSUB = 8
LANES = 16


@functools.cache
def _build_sc_gather(b_items: int, seq: int, nplanes: int, vocab: int):
    info = plsc.get_sparse_core_info()
    nw = info.num_cores * info.num_subcores
    groups = nplanes // N
    dblocks = groups * DIM // SSUB
    s_tiles = seq // SSUB
    b_tiles = b_items // BLK
    units = s_tiles * b_tiles
    assert units % nw == 0
    upw = units // nw
    SQ = 2
    quarters = SSUB // SQ
    chunks = upw * quarters
    nblk = SQ * groups
    mesh = plsc.VectorSubcoreMesh(core_axis_name="c", subcore_axis_name="s")

    @functools.partial(
        pl.kernel,
        mesh=mesh,
        out_type=jax.ShapeDtypeStruct(
            (seq, dblocks, b_tiles, SSUB, BLK), jnp.float32
        ),
        scratch_types=[
            pltpu.VMEM((2, nplanes, SQ, BLK), jnp.int32),
            pltpu.VMEM((nblk, BLK), jnp.int32),
            pltpu.VMEM((2, nblk, BLK, DIM), jnp.float32),
            pltpu.VMEM((2, SQ, dblocks, SSUB, BLK), jnp.float32),
            pltpu.SemaphoreType.DMA,
            pltpu.SemaphoreType.DMA,
            pltpu.SemaphoreType.DMA,
        ],
        compiler_params=pltpu.CompilerParams(
            needs_layout_passes=False, use_tc_tiling_on_sc=False
        ),
    )
    def sc_gather(
        batch_hbm, table_hbm, out_hbm, bv, idx_v, rows_v, tiles_v,
        bsem, gsem, wsem,
    ):
        wid = lax.axis_index("s") * info.num_cores + lax.axis_index("c")
        lane = lax.iota(jnp.int32, LANES)
        lane16 = lane * LANES

        def in_slices(c, buf):
            unit = wid * upw + c // quarters
            tr = unit // b_tiles
            tc = lax.rem(unit, b_tiles)
            q = lax.rem(c, quarters)
            return [
                (
                    batch_hbm.at[w, tr, tc, pl.ds(q * SQ, SQ), :],
                    bv.at[buf, w],
                )
                for w in range(nplanes)
            ]

        for src, dst in in_slices(0, 0):
            pltpu.async_copy(src, dst, bsem)

        def chunk_body(c, carry):
            p = lax.rem(c, 2)
            unit = wid * upw + c // quarters
            tr = unit // b_tiles
            tc = lax.rem(unit, b_tiles)
            q = lax.rem(c, quarters)
            for src, dst in in_slices(c, p):
                pltpu.make_async_copy(src, dst, bsem).wait()

            @pl.when(c >= 2)
            def _():
                for _ in range(SQ * dblocks):
                    pltpu.make_async_copy(
                        tiles_v.at[0, 0, 0], out_hbm.at[0, 0, 0], wsem
                    ).wait()

            for sl in range(SQ):
                for k in range(groups):
                    for i in range(BLK // LANES):
                        csl = pl.ds(i * LANES, LANES)
                        m0 = bv[p, 3 * k, sl, csl]
                        m1 = bv[p, 3 * k + 1, sl, csl]
                        m2 = bv[p, 3 * k + 2, sl, csl]
                        m0 = m0 + (m0 == -1).astype(jnp.int32)
                        m1 = m1 + (m1 == -1).astype(jnp.int32)
                        m2 = m2 + (m2 == -1).astype(jnp.int32)
                        enc = m0 + m1 * 100 + m2 * 10000
                        mx = jnp.maximum(jnp.maximum(m0, m1), m2)
                        idx_v[sl * groups + k, csl] = jnp.where(
                            mx <= MAX_POWER, enc, OVERFLOW
                        )

            gathers = [
                pltpu.async_copy(
                    table_hbm.at[idx_v.at[g]], rows_v.at[p, g], gsem
                )
                for g in range(nblk)
            ]

            @pl.when(c + 1 < chunks)
            def _():
                for src, dst in in_slices(c + 1, lax.rem(c + 1, 2)):
                    pltpu.async_copy(src, dst, bsem)

            for cp in gathers:
                cp.wait()

            for sl in range(SQ):
                for db in range(dblocks):
                    blk = sl * groups + db // 2
                    for ds_ in range(SSUB):
                        e = (db % 2) * SSUB + ds_
                        for i in range(BLK // LANES):
                            val = plsc.load_gather(
                                rows_v.at[p, blk],
                                [lane + i * LANES, jnp.full((LANES,), e, jnp.int32)],
                            )
                            tiles_v[p, sl, db, ds_, pl.ds(i * LANES, LANES)] = val

            for sl in range(SQ):
                for db in range(dblocks):
                    pltpu.async_copy(
                        tiles_v.at[p, sl, db],
                        out_hbm.at[tr * SSUB + q * SQ + sl, db, tc],
                        wsem,
                    )
            return carry

        lax.fori_loop(0, chunks, chunk_body, 0)
        for _ in range(2 * SQ * dblocks):
            pltpu.make_async_copy(
                tiles_v.at[0, 0, 0], out_hbm.at[0, 0, 0], wsem
            ).wait()

    return sc_gather


def kernel(batch, table):
    b, s, w = batch.shape
    b5 = (
        batch.transpose(2, 1, 0)
        .reshape(w, s // SSUB, SSUB, b // BLK, BLK)
        .transpose(0, 1, 3, 2, 4)
    )
    out5 = _build_sc_gather(b, s, w, table.shape[0])(b5, table)
    out = out5.transpose(2, 4, 0, 1, 3).reshape(b, s, (w // N) * DIM)
    return out


# speedup vs baseline: 1.3154x; 1.3154x over previous
import functools

import jax
import jax.numpy as jnp
from jax import lax
from jax.experimental import pallas as pl
from jax.experimental.pallas import tpu as pltpu
from jax.experimental.pallas import tpu_sc as plsc

DIM = 16
MAX_POWER = 99
N = 3
OVERFLOW = (MAX_POWER + 1) ** N
BLK = 128
SSUB = 8
LANES = 16


@functools.cache
def _build_sc_gather(b_items: int, seq: int, nplanes: int, vocab: int):
    info = plsc.get_sparse_core_info()
    nw = info.num_cores * info.num_subcores
    groups = nplanes // N
    dblocks = groups * DIM // SSUB
    s_tiles = seq // SSUB
    b_tiles = b_items // BLK
    units = s_tiles * b_tiles
    assert units % nw == 0
    upw = units // nw
    SQ = 2
    quarters = SSUB // SQ
    chunks = upw * quarters
    nblk = SQ * groups
    mesh = plsc.VectorSubcoreMesh(core_axis_name="c", subcore_axis_name="s")

    @functools.partial(
        pl.kernel,
        mesh=mesh,
        out_type=jax.ShapeDtypeStruct(
            (seq, dblocks, b_tiles, SSUB, BLK), jnp.float32
        ),
        scratch_types=[
            pltpu.VMEM((2, nplanes, SQ, BLK), jnp.int32),
            pltpu.VMEM((nblk, BLK), jnp.int32),
            pltpu.VMEM((2, nblk, BLK, DIM), jnp.float32),
            pltpu.VMEM((2, SQ, dblocks * SSUB, BLK + 1), jnp.float32),
            pltpu.SemaphoreType.DMA,
            pltpu.SemaphoreType.DMA,
            pltpu.SemaphoreType.DMA,
        ],
        compiler_params=pltpu.CompilerParams(
            needs_layout_passes=False, use_tc_tiling_on_sc=False
        ),
    )
    def sc_gather(
        batch_hbm, table_hbm, out_hbm, bv, idx_v, rows_v, tiles_v,
        bsem, gsem, wsem,
    ):
        wid = lax.axis_index("s") * info.num_cores + lax.axis_index("c")
        lane = lax.iota(jnp.int32, LANES)
        lane16 = lane * LANES

        def in_slices(c, buf):
            unit = wid * upw + c // quarters
            tr = unit // b_tiles
            tc = lax.rem(unit, b_tiles)
            q = lax.rem(c, quarters)
            return [
                (
                    batch_hbm.at[w, tr, tc, pl.ds(q * SQ, SQ), :],
                    bv.at[buf, w],
                )
                for w in range(nplanes)
            ]

        for src, dst in in_slices(0, 0):
            pltpu.async_copy(src, dst, bsem)

        def chunk_body(c, carry):
            p = lax.rem(c, 2)
            unit = wid * upw + c // quarters
            tr = unit // b_tiles
            tc = lax.rem(unit, b_tiles)
            q = lax.rem(c, quarters)
            for src, dst in in_slices(c, p):
                pltpu.make_async_copy(src, dst, bsem).wait()

            @pl.when(c >= 2)
            def _():
                for _ in range(SQ * dblocks):
                    pltpu.make_async_copy(
                        tiles_v.at[0, 0, pl.ds(0, SSUB), pl.ds(0, BLK)],
                        out_hbm.at[0, 0, 0],
                        wsem,
                    ).wait()

            for sl in range(SQ):
                for k in range(groups):
                    for i in range(BLK // LANES):
                        csl = pl.ds(i * LANES, LANES)
                        m0 = bv[p, 3 * k, sl, csl]
                        m1 = bv[p, 3 * k + 1, sl, csl]
                        m2 = bv[p, 3 * k + 2, sl, csl]
                        m0 = m0 + (m0 == -1).astype(jnp.int32)
                        m1 = m1 + (m1 == -1).astype(jnp.int32)
                        m2 = m2 + (m2 == -1).astype(jnp.int32)
                        enc = m0 + m1 * 100 + m2 * 10000
                        mx = jnp.maximum(jnp.maximum(m0, m1), m2)
                        idx_v[sl * groups + k, csl] = jnp.where(
                            mx <= MAX_POWER, enc, OVERFLOW
                        )

            gathers = [
                pltpu.async_copy(
                    table_hbm.at[idx_v.at[g]], rows_v.at[p, g], gsem
                )
                for g in range(nblk)
            ]

            @pl.when(c + 1 < chunks)
            def _():
                for src, dst in in_slices(c + 1, lax.rem(c + 1, 2)):
                    pltpu.async_copy(src, dst, bsem)

            for cp in gathers:
                cp.wait()

            for sl in range(SQ):
                for k in range(groups):
                    blk = sl * groups + k
                    d_vec = lane + k * LANES
                    for r in range(BLK):
                        val = rows_v[p, blk, r, :]
                        plsc.store_scatter(
                            tiles_v.at[p, sl],
                            [d_vec, jnp.full((LANES,), r, jnp.int32)],
                            val,
                        )

            for sl in range(SQ):
                for db in range(dblocks):
                    pltpu.async_copy(
                        tiles_v.at[p, sl, pl.ds(db * SSUB, SSUB), pl.ds(0, BLK)],
                        out_hbm.at[tr * SSUB + q * SQ + sl, db, tc],
                        wsem,
                    )
            return carry

        lax.fori_loop(0, chunks, chunk_body, 0)
        for _ in range(2 * SQ * dblocks):
            pltpu.make_async_copy(
                tiles_v.at[0, 0, pl.ds(0, SSUB), pl.ds(0, BLK)],
                out_hbm.at[0, 0, 0],
                wsem,
            ).wait()

    return sc_gather


def kernel(batch, table):
    b, s, w = batch.shape
    b5 = (
        batch.transpose(2, 1, 0)
        .reshape(w, s // SSUB, SSUB, b // BLK, BLK)
        .transpose(0, 1, 3, 2, 4)
    )
    out5 = _build_sc_gather(b, s, w, table.shape[0])(b5, table)
    out = out5.transpose(2, 4, 0, 1, 3).reshape(b, s, (w // N) * DIM)
    return out


# speedup vs baseline: 1.5251x; 1.1595x over previous
import functools

import jax
import jax.numpy as jnp
from jax import lax
from jax.experimental import pallas as pl
from jax.experimental.pallas import tpu as pltpu
from jax.experimental.pallas import tpu_sc as plsc

DIM = 16
MAX_POWER = 99
N = 3
OVERFLOW = (MAX_POWER + 1) ** N
BLK = 128
SSUB = 8
LANES = 16


@functools.cache
def _build_sc_gather(b_items: int, seq: int, nplanes: int, vocab: int):
    info = plsc.get_sparse_core_info()
    nw = info.num_cores * info.num_subcores
    groups = nplanes // N
    dblocks = groups * DIM // SSUB
    s_tiles = seq // SSUB
    b_tiles = b_items // BLK
    units = s_tiles * b_tiles
    assert units % nw == 0
    upw = units // nw
    SQ = 2
    quarters = SSUB // SQ
    chunks = upw * quarters
    assert chunks >= 4
    nblk = SQ * groups
    mesh = plsc.VectorSubcoreMesh(core_axis_name="c", subcore_axis_name="s")

    @functools.partial(
        pl.kernel,
        mesh=mesh,
        out_type=jax.ShapeDtypeStruct(
            (seq, dblocks, b_tiles, SSUB, BLK), jnp.float32
        ),
        scratch_types=[
            pltpu.VMEM((2, nplanes, SQ, BLK), jnp.int32),
            pltpu.VMEM((2, nblk, BLK), jnp.int32),
            pltpu.VMEM((2, nblk, BLK, DIM), jnp.float32),
            pltpu.VMEM((2, SQ, dblocks, SSUB, BLK + 1), jnp.float32),
            pltpu.SemaphoreType.DMA,
            pltpu.SemaphoreType.DMA,
            pltpu.SemaphoreType.DMA,
        ],
        compiler_params=pltpu.CompilerParams(
            needs_layout_passes=False, use_tc_tiling_on_sc=False
        ),
    )
    def sc_gather(
        batch_hbm, table_hbm, out_hbm, bv, idx_v, rows_v, tiles_v,
        bsem, gsem, wsem,
    ):
        wid = lax.axis_index("s") * info.num_cores + lax.axis_index("c")
        lane = lax.iota(jnp.int32, LANES)

        def step_coords(c):
            unit = wid * upw + c // quarters
            tr = unit // b_tiles
            tc = lax.rem(unit, b_tiles)
            q = lax.rem(c, quarters)
            return tr, tc, q

        def in_copy(c, buf):
            tr, tc, q = step_coords(c)
            return pltpu.make_async_copy(
                batch_hbm.at[:, tr, tc, pl.ds(q * SQ, SQ), :],
                bv.at[buf],
                bsem,
            )

        def out_copy(c, buf, sl):
            tr, tc, q = step_coords(c)
            return pltpu.make_async_copy(
                tiles_v.at[buf, sl, :, :, pl.ds(0, BLK)],
                out_hbm.at[tr * SSUB + q * SQ + sl, :, tc, :, :],
                wsem,
            )

        def encode_and_fire(c, p):
            for sl in range(SQ):
                for k in range(groups):
                    for i in range(BLK // LANES):
                        csl = pl.ds(i * LANES, LANES)
                        m0 = bv[p, 3 * k, sl, csl]
                        m1 = bv[p, 3 * k + 1, sl, csl]
                        m2 = bv[p, 3 * k + 2, sl, csl]
                        m0 = m0 + (m0 == -1).astype(jnp.int32)
                        m1 = m1 + (m1 == -1).astype(jnp.int32)
                        m2 = m2 + (m2 == -1).astype(jnp.int32)
                        enc = m0 + m1 * 100 + m2 * 10000
                        mx = jnp.maximum(jnp.maximum(m0, m1), m2)
                        idx_v[p, sl * groups + k, csl] = jnp.where(
                            mx <= MAX_POWER, enc, OVERFLOW
                        )
            for g in range(nblk):
                pltpu.async_copy(
                    table_hbm.at[idx_v.at[p, g]], rows_v.at[p, g], gsem
                )

        def retire(c, p):
            for g in range(nblk):
                pltpu.make_async_copy(
                    table_hbm.at[idx_v.at[p, g]], rows_v.at[p, g], gsem
                ).wait()

            @pl.when(c >= 2)
            def _():
                for _ in range(SQ):
                    out_copy(0, 0, 0).wait()

            for sl in range(SQ):
                for k in range(groups):
                    blk = sl * groups + k
                    dhi = 2 * k + lane // SSUB
                    dlo = lane - (lane // SSUB) * SSUB
                    for r in range(BLK):
                        val = rows_v[p, blk, r, :]
                        plsc.store_scatter(
                            tiles_v.at[p, sl],
                            [dhi, dlo, jnp.full((LANES,), r, jnp.int32)],
                            val,
                        )
                out_copy(c, p, sl).start()

        in_copy(0, 0).start()
        in_copy(0, 0).wait()
        in_copy(1, 1).start()
        encode_and_fire(0, 0)

        def chunk_body(c, carry):
            p = lax.rem(c, 2)
            in_copy(0, p).wait()

            @pl.when(c + 1 < chunks)
            def _():
                in_copy(c + 1, lax.rem(c + 1, 2)).start()

            encode_and_fire(c, p)
            retire(c - 1, 1 - p)
            return carry

        lax.fori_loop(1, chunks, chunk_body, 0)
        retire(chunks - 1, lax.rem(chunks - 1, 2))
        for _ in range(2 * SQ):
            out_copy(0, 0, 0).wait()

    return sc_gather


def kernel(batch, table):
    b, s, w = batch.shape
    b5 = (
        batch.transpose(2, 1, 0)
        .reshape(w, s // SSUB, SSUB, b // BLK, BLK)
        .transpose(0, 1, 3, 2, 4)
    )
    out5 = _build_sc_gather(b, s, w, table.shape[0])(b5, table)
    out = out5.transpose(2, 4, 0, 1, 3).reshape(b, s, (w // N) * DIM)
    return out


# speedup vs baseline: 1.6282x; 1.0676x over previous
import functools

import jax
import jax.numpy as jnp
from jax import lax
from jax.experimental import pallas as pl
from jax.experimental.pallas import tpu as pltpu
from jax.experimental.pallas import tpu_sc as plsc

DIM = 16
MAX_POWER = 99
N = 3
OVERFLOW = (MAX_POWER + 1) ** N
BLK = 128
SSUB = 8
LANES = 16


@functools.cache
def _build_sc_gather(b_items: int, seq: int, nplanes: int, vocab: int):
    info = plsc.get_sparse_core_info()
    nw = info.num_cores * info.num_subcores
    groups = nplanes // N
    dblocks = groups * DIM // SSUB
    s_tiles = seq // SSUB
    b_tiles = b_items // BLK
    units = s_tiles * b_tiles
    assert units % nw == 0
    upw = units // nw
    SQ = 2
    quarters = SSUB // SQ
    chunks = upw * quarters
    assert chunks >= 4
    nblk = SQ * groups
    mesh = plsc.VectorSubcoreMesh(core_axis_name="c", subcore_axis_name="s")

    @functools.partial(
        pl.kernel,
        mesh=mesh,
        out_type=jax.ShapeDtypeStruct(
            (seq, dblocks, b_tiles, SSUB, BLK), jnp.float32
        ),
        scratch_types=[
            pltpu.VMEM((2, nplanes, SQ, BLK), jnp.int32),
            pltpu.VMEM((2, nblk, BLK), jnp.int32),
            pltpu.VMEM((2, nblk, BLK, DIM), jnp.float32),
            pltpu.VMEM((2, SQ, dblocks, SSUB, BLK + 1), jnp.float32),
            pltpu.SemaphoreType.DMA,
            pltpu.SemaphoreType.DMA,
            pltpu.SemaphoreType.DMA,
        ],
        compiler_params=pltpu.CompilerParams(
            needs_layout_passes=False, use_tc_tiling_on_sc=False
        ),
    )
    def sc_gather(
        batch_hbm, table_hbm, out_hbm, bv, idx_v, rows_v, tiles_v,
        bsem, gsem, wsem,
    ):
        wid = lax.axis_index("s") * info.num_cores + lax.axis_index("c")
        lane = lax.iota(jnp.int32, LANES)

        def step_coords(c):
            unit = wid * upw + c // quarters
            tr = unit // b_tiles
            tc = lax.rem(unit, b_tiles)
            q = lax.rem(c, quarters)
            return tr, tc, q

        def in_copy(c, buf):
            tr, tc, q = step_coords(c)
            return pltpu.make_async_copy(
                batch_hbm.at[:, tr, tc, pl.ds(q * SQ, SQ), :],
                bv.at[buf],
                bsem,
            )

        def out_copy(c, buf, sl):
            tr, tc, q = step_coords(c)
            return pltpu.make_async_copy(
                tiles_v.at[buf, sl, :, :, pl.ds(0, BLK)],
                out_hbm.at[tr * SSUB + q * SQ + sl, :, tc, :, :],
                wsem,
            )

        def encode_and_fire(c, p):
            for sl in range(SQ):
                for k in range(groups):
                    for i in range(BLK // LANES):
                        csl = pl.ds(i * LANES, LANES)
                        m0 = bv[p, 3 * k, sl, csl]
                        m1 = bv[p, 3 * k + 1, sl, csl]
                        m2 = bv[p, 3 * k + 2, sl, csl]
                        m0 = m0 + (m0 == -1).astype(jnp.int32)
                        m1 = m1 + (m1 == -1).astype(jnp.int32)
                        m2 = m2 + (m2 == -1).astype(jnp.int32)
                        enc = m0 + m1 * 100 + m2 * 10000
                        mx = jnp.maximum(jnp.maximum(m0, m1), m2)
                        idx_v[p, sl * groups + k, csl] = jnp.where(
                            mx <= MAX_POWER, enc, OVERFLOW
                        )
            for g in range(nblk):
                pltpu.async_copy(
                    table_hbm.at[idx_v.at[p, g]], rows_v.at[p, g], gsem
                )

        def retire(c, p):
            for g in range(nblk):
                pltpu.make_async_copy(
                    table_hbm.at[idx_v.at[p, g]], rows_v.at[p, g], gsem
                ).wait()

            @pl.when(c >= 2)
            def _():
                for _ in range(SQ):
                    out_copy(0, 0, 0).wait()

            for sl in range(SQ):
                for k in range(groups):
                    blk = sl * groups + k
                    dhi = 2 * k + lane // SSUB
                    dlo = lane - (lane // SSUB) * SSUB
                    for r0 in range(0, BLK, 8):
                        vals = [rows_v[p, blk, r0 + j, :] for j in range(8)]
                        for j in range(8):
                            plsc.store_scatter(
                                tiles_v.at[p, sl],
                                [dhi, dlo, jnp.full((LANES,), r0 + j, jnp.int32)],
                                vals[j],
                            )
                out_copy(c, p, sl).start()

        in_copy(0, 0).start()
        in_copy(0, 0).wait()
        in_copy(1, 1).start()
        encode_and_fire(0, 0)

        def chunk_body(c, carry):
            p = lax.rem(c, 2)
            in_copy(0, p).wait()

            @pl.when(c + 1 < chunks)
            def _():
                in_copy(c + 1, lax.rem(c + 1, 2)).start()

            encode_and_fire(c, p)
            retire(c - 1, 1 - p)
            return carry

        lax.fori_loop(1, chunks, chunk_body, 0)
        retire(chunks - 1, lax.rem(chunks - 1, 2))
        for _ in range(2 * SQ):
            out_copy(0, 0, 0).wait()

    return sc_gather


def kernel(batch, table):
    b, s, w = batch.shape
    b5 = (
        batch.transpose(2, 1, 0)
        .reshape(w, s // SSUB, SSUB, b // BLK, BLK)
        .transpose(0, 1, 3, 2, 4)
    )
    out5 = _build_sc_gather(b, s, w, table.shape[0])(b5, table)
    out = out5.transpose(2, 4, 0, 1, 3).reshape(b, s, (w // N) * DIM)
    return out


# speedup vs baseline: 1.6334x; 1.0032x over previous
import functools

import jax
import jax.numpy as jnp
from jax import lax
from jax.experimental import pallas as pl
from jax.experimental.pallas import tpu as pltpu
from jax.experimental.pallas import tpu_sc as plsc

DIM = 16
MAX_POWER = 99
N = 3
OVERFLOW = (MAX_POWER + 1) ** N
BLK = 128
SSUB = 8
LANES = 16


@functools.cache
def _build_sc_gather(b_items: int, seq: int, nplanes: int, vocab: int):
    info = plsc.get_sparse_core_info()
    nw = info.num_cores * info.num_subcores
    groups = nplanes // N
    dblocks = groups * DIM // SSUB
    s_tiles = seq // SSUB
    b_tiles = b_items // BLK
    units = s_tiles * b_tiles
    assert units % nw == 0
    upw = units // nw
    SQ = 2
    quarters = SSUB // SQ
    chunks = upw * quarters
    assert chunks >= 4
    nblk = SQ * groups
    mesh = plsc.VectorSubcoreMesh(core_axis_name="c", subcore_axis_name="s")

    @functools.partial(
        pl.kernel,
        mesh=mesh,
        out_type=jax.ShapeDtypeStruct(
            (seq, dblocks, b_tiles, SSUB, BLK), jnp.float32
        ),
        scratch_types=[
            pltpu.VMEM((2, nplanes, SQ, BLK), jnp.int32),
            pltpu.VMEM((2, nblk, BLK), jnp.int32),
            pltpu.VMEM((2, nblk, BLK, DIM), jnp.float32),
            pltpu.VMEM((2, SQ, dblocks, SSUB, BLK + 1), jnp.float32),
            pltpu.SemaphoreType.DMA,
            pltpu.SemaphoreType.DMA,
            pltpu.SemaphoreType.DMA,
        ],
        compiler_params=pltpu.CompilerParams(
            needs_layout_passes=False, use_tc_tiling_on_sc=False
        ),
    )
    def sc_gather(
        batch_hbm, table_hbm, out_hbm, bv, idx_v, rows_v, tiles_v,
        bsem, gsem, wsem,
    ):
        wid = lax.axis_index("s") * info.num_cores + lax.axis_index("c")
        lane = lax.iota(jnp.int32, LANES)

        def step_coords(c):
            unit = wid * upw + c // quarters
            tr = unit // b_tiles
            tc = lax.rem(unit, b_tiles)
            q = lax.rem(c, quarters)
            return tr, tc, q

        def in_copy(c, buf):
            tr, tc, q = step_coords(c)
            return pltpu.make_async_copy(
                batch_hbm.at[:, tr, tc, pl.ds(q * SQ, SQ), :],
                bv.at[buf],
                bsem,
            )

        def out_copy(c, buf, sl):
            tr, tc, q = step_coords(c)
            return pltpu.make_async_copy(
                tiles_v.at[buf, sl, :, :, pl.ds(0, BLK)],
                out_hbm.at[tr * SSUB + q * SQ + sl, :, tc, :, :],
                wsem,
            )

        def encode_and_fire(c, p):
            for sl in range(SQ):
                for k in range(groups):
                    for i in range(BLK // LANES):
                        csl = pl.ds(i * LANES, LANES)
                        m0 = bv[p, 3 * k, sl, csl]
                        m1 = bv[p, 3 * k + 1, sl, csl]
                        m2 = bv[p, 3 * k + 2, sl, csl]
                        m0 = m0 + (m0 == -1).astype(jnp.int32)
                        m1 = m1 + (m1 == -1).astype(jnp.int32)
                        m2 = m2 + (m2 == -1).astype(jnp.int32)
                        enc = m0 + m1 * 100 + m2 * 10000
                        mx = jnp.maximum(jnp.maximum(m0, m1), m2)
                        idx_v[p, sl * groups + k, csl] = jnp.where(
                            mx <= MAX_POWER, enc, OVERFLOW
                        )
            for g in range(nblk):
                pltpu.async_copy(
                    table_hbm.at[idx_v.at[p, g]], rows_v.at[p, g], gsem
                )

        def retire(c, p, rzero):
            for g in range(nblk):
                pltpu.make_async_copy(
                    table_hbm.at[idx_v.at[p, g]], rows_v.at[p, g], gsem
                ).wait()

            @pl.when(c >= 2)
            def _():
                for _ in range(SQ):
                    out_copy(0, 0, 0).wait()

            for sl in range(SQ):
                for k in range(groups):
                    blk = sl * groups + k
                    dhi = 2 * k + lane // SSUB
                    dlo = lane - (lane // SSUB) * SSUB
                    rv = rzero
                    for r0 in range(0, BLK, 8):
                        vals = [rows_v[p, blk, r0 + j, :] for j in range(8)]
                        for j in range(8):
                            plsc.store_scatter(
                                tiles_v.at[p, sl],
                                [dhi, dlo, rv],
                                vals[j],
                            )
                            rv = rv + 1
                out_copy(c, p, sl).start()

        in_copy(0, 0).start()
        in_copy(0, 0).wait()
        in_copy(1, 1).start()
        encode_and_fire(0, 0)

        def chunk_body(c, carry):
            p = lax.rem(c, 2)
            in_copy(0, p).wait()

            @pl.when(c + 1 < chunks)
            def _():
                in_copy(c + 1, lax.rem(c + 1, 2)).start()

            encode_and_fire(c, p)
            retire(c - 1, 1 - p, rzero)
            return carry

        rzero = jnp.full((LANES,), 0, jnp.int32) + wid * 0
        lax.fori_loop(1, chunks, chunk_body, 0)
        retire(chunks - 1, lax.rem(chunks - 1, 2), rzero)
        for _ in range(2 * SQ):
            out_copy(0, 0, 0).wait()

    return sc_gather


def kernel(batch, table):
    b, s, w = batch.shape
    b5 = (
        batch.transpose(2, 1, 0)
        .reshape(w, s // SSUB, SSUB, b // BLK, BLK)
        .transpose(0, 1, 3, 2, 4)
    )
    out5 = _build_sc_gather(b, s, w, table.shape[0])(b5, table)
    out = out5.transpose(2, 4, 0, 1, 3).reshape(b, s, (w // N) * DIM)
    return out
